# Initial kernel scaffold; baseline (speedup 1.0000x reference)
import jax
import jax.numpy as jnp
from jax.experimental import pallas as pl


def kernel(features, pos):
    raise NotImplementedError("write your pallas kernel here")


# speedup vs baseline: 22.0032x; 22.0032x over previous
import jax
import jax.numpy as jnp
from jax.experimental import pallas as pl

_SUB = 5
_K = 8
_ROWS = 512
_MPAD = 832


def _knn_body(x_ref, sqr_ref, sqc_ref, out_ref):
    x = x_ref[0]
    sq_row = sqr_ref[0]
    sq_col = sqc_ref[0]
    xr = x_ref[0, pl.ds(pl.program_id(1) * _ROWS, _ROWS), :]
    n = x.shape[0]
    cols = jax.lax.broadcasted_iota(jnp.int32, (_ROWS, n), 1)
    inf = jnp.float32(jnp.inf)
    g = jax.lax.dot_general(
        xr, x, (((1,), (1,)), ((), ())),
        preferred_element_type=jnp.float32)
    d2 = (sq_row + sq_col) - 2.0 * g
    work = jnp.maximum(d2, 0.0)
    vals = []
    for k in range(_K):
        m = jnp.min(work, axis=1, keepdims=True)
        vals.append(jnp.sqrt(m))
        if k < _K - 1:
            eq = work == m
            jmin = jnp.min(jnp.where(eq, cols, n), axis=1, keepdims=True)
            work = jnp.where(cols == jmin, inf, work)
    out_ref[0] = jnp.concatenate(vals, axis=1)


def _select_body(denr_ref, denc_ref, fp_ref, out_ref):
    den_col = denc_ref[0]
    fp = fp_ref[0]
    n = fp.shape[0]
    icol = jax.lax.broadcasted_iota(jnp.int32, (_ROWS, n), 1)
    rank = jnp.zeros((1, n), jnp.int32)
    for t in range(n // _ROWS):
        dj = denr_ref[0, t * _ROWS:(t + 1) * _ROWS, :]
        jrow = t * _ROWS + jax.lax.broadcasted_iota(jnp.int32, (_ROWS, n), 0)
        gt = (dj > den_col).astype(jnp.int32)
        eq = ((dj == den_col) & (jrow < icol)).astype(jnp.int32)
        rank = rank + jnp.sum(gt + eq, axis=0, keepdims=True)
    mrow = jax.lax.broadcasted_iota(jnp.int32, (_MPAD, n), 0)
    p = (mrow == rank).astype(jnp.float32)
    out_ref[0] = jnp.dot(p, fp, preferred_element_type=jnp.float32,
                         precision=jax.lax.Precision.HIGHEST)


def kernel(features, pos):
    B, N, C = features.shape
    M = N // _SUB
    sq = jnp.sum(features * features, axis=-1)
    sq_row = sq.reshape(B, N, 1)
    sq_col = sq.reshape(B, 1, N)
    knn = pl.pallas_call(
        _knn_body,
        grid=(B, N // _ROWS),
        in_specs=[
            pl.BlockSpec((1, N, C), lambda b, t: (b, 0, 0)),
            pl.BlockSpec((1, _ROWS, 1), lambda b, t: (b, t, 0)),
            pl.BlockSpec((1, 1, N), lambda b, t: (b, 0, 0)),
        ],
        out_specs=pl.BlockSpec((1, _ROWS, _K), lambda b, t: (b, t, 0)),
        out_shape=jax.ShapeDtypeStruct((B, N, _K), jnp.float32),
    )(features, sq_row, sq_col)
    knn = jax.lax.optimization_barrier(knn)
    density = jnp.mean(knn, axis=-1)
    density = jax.lax.optimization_barrier(density)
    fp = jnp.concatenate(
        [features, pos, jnp.zeros((B, N, 64 - C - 3), jnp.float32)], axis=-1)
    out = pl.pallas_call(
        _select_body,
        grid=(B,),
        in_specs=[
            pl.BlockSpec((1, N, 1), lambda b: (b, 0, 0)),
            pl.BlockSpec((1, 1, N), lambda b: (b, 0, 0)),
            pl.BlockSpec((1, N, 64), lambda b: (b, 0, 0)),
        ],
        out_specs=pl.BlockSpec((1, _MPAD, 64), lambda b: (b, 0, 0)),
        out_shape=jax.ShapeDtypeStruct((B, _MPAD, 64), jnp.float32),
    )(density.reshape(B, N, 1), density.reshape(B, 1, N), fp)
    return out[:, :M, :C], out[:, :M, C:C + 3]


# speedup vs baseline: 22.4969x; 1.0224x over previous
import functools

import jax
import jax.numpy as jnp
from jax import lax
from jax.experimental import pallas as pl
from jax.experimental.pallas import tpu as pltpu, tpu_sc as plsc

_SUB = 5
_K = 8
_ROWS = 512
_MPAD = 832
_NC = 2
_NS = 16
_NW = _NC * _NS


def _knn_body(x_ref, sqr_ref, sqc_ref, out_ref):
    x = x_ref[0]
    sq_row = sqr_ref[0]
    sq_col = sqc_ref[0]
    xr = x_ref[0, pl.ds(pl.program_id(1) * _ROWS, _ROWS), :]
    n = x.shape[0]
    cols = jax.lax.broadcasted_iota(jnp.int32, (_ROWS, n), 1)
    inf = jnp.float32(jnp.inf)
    g = jax.lax.dot_general(
        xr, x, (((1,), (1,)), ((), ())),
        preferred_element_type=jnp.float32)
    d2 = (sq_row + sq_col) - 2.0 * g
    work = jnp.maximum(d2, 0.0)
    vals = []
    for k in range(_K):
        m = jnp.min(work, axis=1, keepdims=True)
        vals.append(jnp.sqrt(m))
        if k < _K - 1:
            eq = work == m
            jmin = jnp.min(jnp.where(eq, cols, n), axis=1, keepdims=True)
            work = jnp.where(cols == jmin, inf, work)
    out_ref[0] = jnp.concatenate(vals, axis=1)


def _rank_inds_body(denr_ref, denc_ref, out_ref):
    den_col = denc_ref[0]
    n = den_col.shape[1]
    icol = jax.lax.broadcasted_iota(jnp.int32, (_ROWS, n), 1)
    rank = jnp.zeros((1, n), jnp.int32)
    for t in range(n // _ROWS):
        dj = denr_ref[0, t * _ROWS:(t + 1) * _ROWS, :]
        jrow = t * _ROWS + jax.lax.broadcasted_iota(jnp.int32, (_ROWS, n), 0)
        gt = (dj > den_col).astype(jnp.int32)
        eq = ((dj == den_col) & (jrow < icol)).astype(jnp.int32)
        rank = rank + jnp.sum(gt + eq, axis=0, keepdims=True)
    mrow = jax.lax.broadcasted_iota(jnp.int32, (_MPAD, n), 0)
    p = (mrow == rank).astype(jnp.float32)
    iota_col = jax.lax.broadcasted_iota(jnp.int32, (n, 1), 0).astype(jnp.float32)
    out_ref[0] = jnp.dot(p, iota_col, preferred_element_type=jnp.float32,
                         precision=jax.lax.Precision.HIGHEST)


def _make_sc_gather(v, d, b_total):
    b_per_w = b_total // _NW
    mesh = plsc.VectorSubcoreMesh(core_axis_name="c", subcore_axis_name="s")

    @functools.partial(
        pl.kernel, mesh=mesh,
        out_type=jax.ShapeDtypeStruct((b_total, d), jnp.float32),
        scratch_types=[
            pltpu.VMEM((b_per_w,), jnp.int32),
            pltpu.VMEM((b_per_w, d), jnp.float32),
            pltpu.SemaphoreType.DMA,
        ],
    )
    def gather_k(table_hbm, idx_hbm, out_hbm, idx_v, rows_v, sem):
        wid = lax.axis_index("s") * _NC + lax.axis_index("c")
        base = wid * b_per_w
        pltpu.sync_copy(idx_hbm.at[pl.ds(base, b_per_w)], idx_v)
        pltpu.async_copy(table_hbm.at[idx_v], rows_v, sem).wait()
        pltpu.sync_copy(rows_v, out_hbm.at[pl.ds(base, b_per_w)])

    return gather_k


def kernel(features, pos):
    B, N, C = features.shape
    M = N // _SUB
    sq = jnp.sum(features * features, axis=-1)
    knn = pl.pallas_call(
        _knn_body,
        grid=(B, N // _ROWS),
        in_specs=[
            pl.BlockSpec((1, N, C), lambda b, t: (b, 0, 0)),
            pl.BlockSpec((1, _ROWS, 1), lambda b, t: (b, t, 0)),
            pl.BlockSpec((1, 1, N), lambda b, t: (b, 0, 0)),
        ],
        out_specs=pl.BlockSpec((1, _ROWS, _K), lambda b, t: (b, t, 0)),
        out_shape=jax.ShapeDtypeStruct((B, N, _K), jnp.float32),
    )(features, sq.reshape(B, N, 1), sq.reshape(B, 1, N))
    knn = jax.lax.optimization_barrier(knn)
    density = jnp.mean(knn, axis=-1)
    density = jax.lax.optimization_barrier(density)
    inds_f = pl.pallas_call(
        _rank_inds_body,
        grid=(B,),
        in_specs=[
            pl.BlockSpec((1, N, 1), lambda b: (b, 0, 0)),
            pl.BlockSpec((1, 1, N), lambda b: (b, 0, 0)),
        ],
        out_specs=pl.BlockSpec((1, _MPAD, 1), lambda b: (b, 0, 0)),
        out_shape=jax.ShapeDtypeStruct((B, _MPAD, 1), jnp.float32),
    )(density.reshape(B, N, 1), density.reshape(B, 1, N))
    inds = inds_f.reshape(B, _MPAD).astype(jnp.int32)
    flat_idx = (inds + jnp.arange(B, dtype=jnp.int32)[:, None] * N).reshape(-1)
    fp = jnp.concatenate(
        [features, pos, jnp.zeros((B, N, 128 - C - 3), jnp.float32)],
        axis=-1).reshape(B * N, 128)
    out = _make_sc_gather(B * N, 128, B * _MPAD)(fp, flat_idx)
    out = out.reshape(B, _MPAD, 128)
    return out[:, :M, :C], out[:, :M, C:C + 3]


# speedup vs baseline: 23.0078x; 1.0227x over previous
import functools

import jax
import jax.numpy as jnp
from jax import lax
from jax.experimental import pallas as pl
from jax.experimental.pallas import tpu as pltpu, tpu_sc as plsc

_SUB = 5
_K = 8
_ROWS = 512
_MPAD = 832
_NC = 2
_NS = 16
_NW = _NC * _NS


def _knn_body(x_ref, sqr_ref, sqc_ref, out_ref):
    x = x_ref[0]
    sq_row = sqr_ref[0]
    sq_col = sqc_ref[0]
    xr = x_ref[0, pl.ds(pl.program_id(1) * _ROWS, _ROWS), :]
    n = x.shape[0]
    cols = jax.lax.broadcasted_iota(jnp.int32, (_ROWS, n), 1)
    inf = jnp.float32(jnp.inf)
    g = jax.lax.dot_general(
        xr, x, (((1,), (1,)), ((), ())),
        preferred_element_type=jnp.float32)
    d2 = (sq_row + sq_col) - 2.0 * g
    work = jnp.maximum(d2, 0.0)
    rowg = pl.program_id(1) * _ROWS + jax.lax.broadcasted_iota(
        jnp.int32, (_ROWS, n), 0)
    diag = cols == rowg
    dval = jnp.sum(jnp.where(diag, work, 0.0), axis=1, keepdims=True)
    work = jnp.where(diag, inf, work)
    vals = [jnp.sqrt(dval)]
    for k in range(1, _K):
        m = jnp.min(work, axis=1, keepdims=True)
        vals.append(jnp.sqrt(m))
        if k < _K - 1:
            eq = work == m
            jmin = jnp.min(jnp.where(eq, cols, n), axis=1, keepdims=True)
            work = jnp.where(cols == jmin, inf, work)
    out_ref[0] = jnp.concatenate(vals, axis=1)


def _rank_inds_body(denr_ref, denc_ref, out_ref):
    den_col = denc_ref[0]
    n = den_col.shape[1]
    icol = jax.lax.broadcasted_iota(jnp.int32, (_ROWS, n), 1)
    rank = jnp.zeros((1, n), jnp.int32)
    for t in range(n // _ROWS):
        dj = denr_ref[0, t * _ROWS:(t + 1) * _ROWS, :]
        jrow = t * _ROWS + jax.lax.broadcasted_iota(jnp.int32, (_ROWS, n), 0)
        gt = (dj > den_col).astype(jnp.int32)
        eq = ((dj == den_col) & (jrow < icol)).astype(jnp.int32)
        rank = rank + jnp.sum(gt + eq, axis=0, keepdims=True)
    mrow = jax.lax.broadcasted_iota(jnp.int32, (_MPAD, n), 0)
    p = (mrow == rank).astype(jnp.float32)
    iota_col = jax.lax.broadcasted_iota(jnp.int32, (n, 1), 0).astype(jnp.float32)
    out_ref[0] = jnp.dot(p, iota_col, preferred_element_type=jnp.float32,
                         precision=jax.lax.Precision.HIGHEST)


def _make_sc_gather(v, d, b_total):
    b_per_w = b_total // _NW
    mesh = plsc.VectorSubcoreMesh(core_axis_name="c", subcore_axis_name="s")

    @functools.partial(
        pl.kernel, mesh=mesh,
        out_type=jax.ShapeDtypeStruct((b_total, d), jnp.float32),
        scratch_types=[
            pltpu.VMEM((b_per_w,), jnp.int32),
            pltpu.VMEM((b_per_w, d), jnp.float32),
            pltpu.SemaphoreType.DMA,
        ],
    )
    def gather_k(table_hbm, idx_hbm, out_hbm, idx_v, rows_v, sem):
        wid = lax.axis_index("s") * _NC + lax.axis_index("c")
        base = wid * b_per_w
        pltpu.sync_copy(idx_hbm.at[pl.ds(base, b_per_w)], idx_v)
        pltpu.async_copy(table_hbm.at[idx_v], rows_v, sem).wait()
        pltpu.sync_copy(rows_v, out_hbm.at[pl.ds(base, b_per_w)])

    return gather_k


def kernel(features, pos):
    B, N, C = features.shape
    M = N // _SUB
    sq = jnp.sum(features * features, axis=-1)
    knn = pl.pallas_call(
        _knn_body,
        grid=(B, N // _ROWS),
        in_specs=[
            pl.BlockSpec((1, N, C), lambda b, t: (b, 0, 0)),
            pl.BlockSpec((1, _ROWS, 1), lambda b, t: (b, t, 0)),
            pl.BlockSpec((1, 1, N), lambda b, t: (b, 0, 0)),
        ],
        out_specs=pl.BlockSpec((1, _ROWS, _K), lambda b, t: (b, t, 0)),
        out_shape=jax.ShapeDtypeStruct((B, N, _K), jnp.float32),
    )(features, sq.reshape(B, N, 1), sq.reshape(B, 1, N))
    knn = jax.lax.optimization_barrier(knn)
    density = jnp.mean(knn, axis=-1)
    density = jax.lax.optimization_barrier(density)
    inds_f = pl.pallas_call(
        _rank_inds_body,
        grid=(B,),
        in_specs=[
            pl.BlockSpec((1, N, 1), lambda b: (b, 0, 0)),
            pl.BlockSpec((1, 1, N), lambda b: (b, 0, 0)),
        ],
        out_specs=pl.BlockSpec((1, _MPAD, 1), lambda b: (b, 0, 0)),
        out_shape=jax.ShapeDtypeStruct((B, _MPAD, 1), jnp.float32),
    )(density.reshape(B, N, 1), density.reshape(B, 1, N))
    inds = inds_f.reshape(B, _MPAD).astype(jnp.int32)
    flat_idx = (inds + jnp.arange(B, dtype=jnp.int32)[:, None] * N).reshape(-1)
    fp = jnp.concatenate(
        [features, pos, jnp.zeros((B, N, 128 - C - 3), jnp.float32)],
        axis=-1).reshape(B * N, 128)
    out = _make_sc_gather(B * N, 128, B * _MPAD)(fp, flat_idx)
    out = out.reshape(B, _MPAD, 128)
    return out[:, :M, :C], out[:, :M, C:C + 3]
